# baseline (speedup 1.0000x reference)
import functools

import jax
import jax.numpy as jnp
from jax.experimental import pallas as pl
from jax.experimental.pallas import tpu as pltpu
from jax.experimental.pallas import tpu_sc as plsc


def _make_enc_body(P, H, C, CHPAD):
    half = H // 2
    quarter = half // 2

    def body(ch_ref, pos_ref, che_ref):
        j = jax.lax.broadcasted_iota(jnp.int32, (1, quarter), 1)
        omega = 1.0 / (10000.0 ** (j.astype(jnp.float32) / float(quarter)))

        sub = 1
        while sub * sub < P:
            sub *= 2
        if sub * sub == P:
            t = jax.lax.broadcasted_iota(jnp.int32, (sub, quarter), 0)
            t = t.astype(jnp.float32)
            ang_a = (t * float(sub)) * omega
            ang_b = t * omega
            sa_all, ca_all = jnp.sin(ang_a), jnp.cos(ang_a)
            sb, cb = jnp.sin(ang_b), jnp.cos(ang_b)
            for a in range(sub):
                sa = sa_all[a:a + 1, :]
                ca = ca_all[a:a + 1, :]
                rows = pl.ds(a * sub, sub)
                pos_ref[rows, :quarter] = sa * cb + ca * sb
                pos_ref[rows, quarter:] = ca * cb - sa * sb
        else:
            p = jax.lax.broadcasted_iota(jnp.int32, (P, quarter), 0)
            ang = p.astype(jnp.float32) * omega
            pos_ref[:, :quarter] = jnp.sin(ang)
            pos_ref[:, quarter:] = jnp.cos(ang)

        for c in range(C):
            ch = ch_ref[c].astype(jnp.float32)
            ang_c = ch * omega
            che_ref[c:c + 1, :quarter] = jnp.sin(ang_c)
            che_ref[c:c + 1, quarter:] = jnp.cos(ang_c)
        if CHPAD > C:
            che_ref[pl.ds(C, CHPAD - C), :] = jnp.zeros(
                (CHPAD - C, half), jnp.float32)

    return body


def _enc_tables(channels, P, H, C):
    CHPAD = 16
    half = H // 2
    return pl.pallas_call(
        _make_enc_body(P, H, C, CHPAD),
        in_specs=[pl.BlockSpec(memory_space=pltpu.SMEM)],
        out_shape=[
            jax.ShapeDtypeStruct((P, half), jnp.float32),
            jax.ShapeDtypeStruct((CHPAD, half), jnp.float32),
        ],
    )(channels)


def _sc_add(xf, pos_enc, ch_enc, P, C):
    R, H = xf.shape
    half = H // 2
    info = plsc.get_sparse_core_info()
    NW = info.num_cores * info.num_subcores
    PW = P // NW
    S = R // P
    mesh = plsc.VectorSubcoreMesh(core_axis_name="c", subcore_axis_name="s")

    @functools.partial(
        pl.kernel,
        out_type=jax.ShapeDtypeStruct((R, H), jnp.float32),
        mesh=mesh,
        scratch_types=[
            pltpu.VMEM((PW, half), jnp.float32),
            pltpu.VMEM((16, half), jnp.float32),
            pltpu.VMEM((PW, H), jnp.float32),
        ],
    )
    def run(x_hbm, pos_hbm, ch_hbm, out_hbm, pos_v, ch_v, xt):
        cid = jax.lax.axis_index("c")
        sid = jax.lax.axis_index("s")
        wid = sid * info.num_cores + cid
        p0 = wid * PW
        pltpu.sync_copy(pos_hbm.at[pl.ds(p0, PW)], pos_v)
        pltpu.sync_copy(ch_hbm, ch_v)

        def seg(s, carry):
            base = s * P + p0
            pltpu.sync_copy(x_hbm.at[pl.ds(base, PW)], xt)
            c = jax.lax.rem(s, C)

            def row(r, carry2):
                for k in range(half // 16):
                    sl = pl.ds(k * 16, 16)
                    xt[r, sl] = xt[r, sl] + ch_v[c, sl]
                for k in range(half // 16):
                    sl_x = pl.ds(half + k * 16, 16)
                    sl_e = pl.ds(k * 16, 16)
                    xt[r, sl_x] = xt[r, sl_x] + pos_v[r, sl_e]
                return carry2

            jax.lax.fori_loop(0, PW, row, 0)
            pltpu.sync_copy(xt, out_hbm.at[pl.ds(base, PW)])
            return carry

        jax.lax.fori_loop(0, S, seg, 0)

    return run(xf, pos_enc, ch_enc)


def kernel(x, channels):
    B, CP, H = x.shape
    C = channels.shape[0]
    if not C:
        return x
    P = CP // C
    xf = x.reshape(B * CP, H)
    pos_enc, ch_enc = _enc_tables(channels, P, H, C)
    out = _sc_add(xf, pos_enc, ch_enc, P, C)
    return out.reshape(x.shape)


# speedup vs baseline: 1.3905x; 1.3905x over previous
import functools

import jax
import jax.numpy as jnp
from jax.experimental import pallas as pl
from jax.experimental.pallas import tpu as pltpu
from jax.experimental.pallas import tpu_sc as plsc


def _make_enc_body(P, H, C, CHPAD):
    half = H // 2
    quarter = half // 2

    def body(ch_ref, pos_ref, che_ref):
        j = jax.lax.broadcasted_iota(jnp.int32, (1, quarter), 1)
        omega = 1.0 / (10000.0 ** (j.astype(jnp.float32) / float(quarter)))

        sub = 1
        while sub * sub < P:
            sub *= 2
        if sub * sub == P:
            t = jax.lax.broadcasted_iota(jnp.int32, (sub, quarter), 0)
            t = t.astype(jnp.float32)
            ang_a = (t * float(sub)) * omega
            ang_b = t * omega
            sa_all, ca_all = jnp.sin(ang_a), jnp.cos(ang_a)
            sb, cb = jnp.sin(ang_b), jnp.cos(ang_b)
            for a in range(sub):
                sa = sa_all[a:a + 1, :]
                ca = ca_all[a:a + 1, :]
                rows = pl.ds(a * sub, sub)
                pos_ref[rows, :quarter] = sa * cb + ca * sb
                pos_ref[rows, quarter:] = ca * cb - sa * sb
        else:
            p = jax.lax.broadcasted_iota(jnp.int32, (P, quarter), 0)
            ang = p.astype(jnp.float32) * omega
            pos_ref[:, :quarter] = jnp.sin(ang)
            pos_ref[:, quarter:] = jnp.cos(ang)

        for c in range(C):
            ch = ch_ref[c].astype(jnp.float32)
            ang_c = ch * omega
            che_ref[c:c + 1, :quarter] = jnp.sin(ang_c)
            che_ref[c:c + 1, quarter:] = jnp.cos(ang_c)
        if CHPAD > C:
            che_ref[pl.ds(C, CHPAD - C), :] = jnp.zeros(
                (CHPAD - C, half), jnp.float32)

    return body


def _enc_tables(channels, P, H, C):
    CHPAD = 16
    half = H // 2
    return pl.pallas_call(
        _make_enc_body(P, H, C, CHPAD),
        in_specs=[pl.BlockSpec(memory_space=pltpu.SMEM)],
        out_shape=[
            jax.ShapeDtypeStruct((P, half), jnp.float32),
            jax.ShapeDtypeStruct((CHPAD, half), jnp.float32),
        ],
    )(channels)


def _sc_add(xf, pos_enc, ch_enc, P, C):
    R, H = xf.shape
    half = H // 2
    info = plsc.get_sparse_core_info()
    NW = info.num_cores * info.num_subcores
    PW = P // NW
    S = R // P
    mesh = plsc.VectorSubcoreMesh(core_axis_name="c", subcore_axis_name="s")

    NBUF = 4
    assert S % NBUF == 0

    @functools.partial(
        pl.kernel,
        out_type=jax.ShapeDtypeStruct((R, H), jnp.float32),
        mesh=mesh,
        scratch_types=[
            pltpu.VMEM((PW, half), jnp.float32),
            pltpu.VMEM((16, half), jnp.float32),
            pltpu.VMEM((NBUF, PW, H), jnp.float32),
        ] + [pltpu.SemaphoreType.DMA] * (2 * NBUF),
    )
    def run(x_hbm, pos_hbm, ch_hbm, out_hbm, pos_v, ch_v, xt, *sems):
        in_sem, out_sem = sems[:NBUF], sems[NBUF:]
        cid = jax.lax.axis_index("c")
        sid = jax.lax.axis_index("s")
        wid = sid * info.num_cores + cid
        p0 = wid * PW
        pltpu.sync_copy(pos_hbm.at[pl.ds(p0, PW)], pos_v)
        pltpu.sync_copy(ch_hbm, ch_v)

        def in_copy(s, b):
            return pltpu.make_async_copy(
                x_hbm.at[pl.ds(s * P + p0, PW)], xt.at[b], in_sem[b])

        def out_copy(s, b):
            return pltpu.make_async_copy(
                xt.at[b], out_hbm.at[pl.ds(s * P + p0, PW)], out_sem[b])

        for b in range(NBUF - 2):
            in_copy(b, b).start()

        def outer(g, carry):
            for b in range(NBUF):
                s = g + b
                nxt = s + 2
                bn = (b + 2) % NBUF

                @pl.when(s >= 2)
                def _():
                    out_copy(s - 2, bn).wait()

                @pl.when(nxt < S)
                def _():
                    in_copy(nxt, bn).start()

                in_copy(s, b).wait()
                c = jax.lax.rem(s, C)

                def row(r, carry2):
                    for k in range(half // 16):
                        sl = pl.ds(k * 16, 16)
                        xt[b, r, sl] = xt[b, r, sl] + ch_v[c, sl]
                    for k in range(half // 16):
                        sl_x = pl.ds(half + k * 16, 16)
                        sl_e = pl.ds(k * 16, 16)
                        xt[b, r, sl_x] = xt[b, r, sl_x] + pos_v[r, sl_e]
                    return carry2

                jax.lax.fori_loop(0, PW, row, 0)
                out_copy(s, b).start()
            return carry

        jax.lax.fori_loop(0, S // NBUF, lambda i, c: outer(i * NBUF, c), 0)
        for s_tail in (S - 2, S - 1):
            out_copy(s_tail, s_tail % NBUF).wait()

    return run(xf, pos_enc, ch_enc)


def kernel(x, channels):
    B, CP, H = x.shape
    C = channels.shape[0]
    if not C:
        return x
    P = CP // C
    xf = x.reshape(B * CP, H)
    pos_enc, ch_enc = _enc_tables(channels, P, H, C)
    out = _sc_add(xf, pos_enc, ch_enc, P, C)
    return out.reshape(x.shape)


# speedup vs baseline: 3.6367x; 2.6154x over previous
import functools

import jax
import jax.numpy as jnp
from jax.experimental import pallas as pl
from jax.experimental.pallas import tpu as pltpu
from jax.experimental.pallas import tpu_sc as plsc


def _make_enc_body(P, H, C, CHPAD):
    half = H // 2
    quarter = half // 2

    def body(ch_ref, pos_ref, che_ref):
        j = jax.lax.broadcasted_iota(jnp.int32, (1, quarter), 1)
        omega = 1.0 / (10000.0 ** (j.astype(jnp.float32) / float(quarter)))

        sub = 1
        while sub * sub < P:
            sub *= 2
        if sub * sub == P:
            t = jax.lax.broadcasted_iota(jnp.int32, (sub, quarter), 0)
            t = t.astype(jnp.float32)
            ang_a = (t * float(sub)) * omega
            ang_b = t * omega
            sa_all, ca_all = jnp.sin(ang_a), jnp.cos(ang_a)
            sb, cb = jnp.sin(ang_b), jnp.cos(ang_b)
            for a in range(sub):
                sa = sa_all[a:a + 1, :]
                ca = ca_all[a:a + 1, :]
                rows = pl.ds(a * sub, sub)
                pos_ref[rows, :quarter] = sa * cb + ca * sb
                pos_ref[rows, quarter:] = ca * cb - sa * sb
        else:
            p = jax.lax.broadcasted_iota(jnp.int32, (P, quarter), 0)
            ang = p.astype(jnp.float32) * omega
            pos_ref[:, :quarter] = jnp.sin(ang)
            pos_ref[:, quarter:] = jnp.cos(ang)

        for c in range(C):
            ch = ch_ref[c].astype(jnp.float32)
            ang_c = ch * omega
            che_ref[c:c + 1, :quarter] = jnp.sin(ang_c)
            che_ref[c:c + 1, quarter:] = jnp.cos(ang_c)
        if CHPAD > C:
            che_ref[pl.ds(C, CHPAD - C), :] = jnp.zeros(
                (CHPAD - C, half), jnp.float32)

    return body


def _enc_tables(channels, P, H, C):
    CHPAD = 16
    half = H // 2
    return pl.pallas_call(
        _make_enc_body(P, H, C, CHPAD),
        in_specs=[pl.BlockSpec(memory_space=pltpu.SMEM)],
        out_shape=[
            jax.ShapeDtypeStruct((P, half), jnp.float32),
            jax.ShapeDtypeStruct((CHPAD, half), jnp.float32),
        ],
    )(channels)


def _sc_add(xf, pos_enc, ch_enc, P, C):
    R, H = xf.shape
    half = H // 2
    info = plsc.get_sparse_core_info()
    NW = info.num_cores * info.num_subcores
    PW = P // NW
    S = R // P
    mesh = plsc.VectorSubcoreMesh(core_axis_name="c", subcore_axis_name="s")

    NBUF = 4
    assert S % NBUF == 0

    @functools.partial(
        pl.kernel,
        out_type=jax.ShapeDtypeStruct((R, H), jnp.float32),
        mesh=mesh,
        scratch_types=[
            pltpu.VMEM((PW, half), jnp.float32),
            pltpu.VMEM((16, half), jnp.float32),
            pltpu.VMEM((NBUF, PW, H), jnp.float32),
        ] + [pltpu.SemaphoreType.DMA] * (2 * NBUF),
    )
    def run(x_hbm, pos_hbm, ch_hbm, out_hbm, pos_v, ch_v, xt, *sems):
        in_sem, out_sem = sems[:NBUF], sems[NBUF:]
        cid = jax.lax.axis_index("c")
        sid = jax.lax.axis_index("s")
        wid = sid * info.num_cores + cid
        p0 = wid * PW
        pltpu.sync_copy(pos_hbm.at[pl.ds(p0, PW)], pos_v)
        pltpu.sync_copy(ch_hbm, ch_v)

        def in_copy(s, b):
            return pltpu.make_async_copy(
                x_hbm.at[pl.ds(s * P + p0, PW)], xt.at[b], in_sem[b])

        def out_copy(s, b):
            return pltpu.make_async_copy(
                xt.at[b], out_hbm.at[pl.ds(s * P + p0, PW)], out_sem[b])

        for b in range(NBUF - 2):
            in_copy(b, b).start()

        def outer(g, carry):
            for b in range(NBUF):
                s = g + b
                nxt = s + 2
                bn = (b + 2) % NBUF

                @pl.when(s >= 2)
                def _():
                    out_copy(s - 2, bn).wait()

                @pl.when(nxt < S)
                def _():
                    in_copy(nxt, bn).start()

                in_copy(s, b).wait()
                c = jax.lax.rem(s, C)

                @plsc.parallel_loop(0, PW, unroll=2)
                def _row(r):
                    for k in range(half // 16):
                        sl = pl.ds(k * 16, 16)
                        xt[b, r, sl] = xt[b, r, sl] + ch_v[c, sl]
                    for k in range(half // 16):
                        sl_x = pl.ds(half + k * 16, 16)
                        sl_e = pl.ds(k * 16, 16)
                        xt[b, r, sl_x] = xt[b, r, sl_x] + pos_v[r, sl_e]

                out_copy(s, b).start()
            return carry

        jax.lax.fori_loop(0, S // NBUF, lambda i, c: outer(i * NBUF, c), 0)
        for s_tail in (S - 2, S - 1):
            out_copy(s_tail, s_tail % NBUF).wait()

    return run(xf, pos_enc, ch_enc)


def kernel(x, channels):
    B, CP, H = x.shape
    C = channels.shape[0]
    if not C:
        return x
    P = CP // C
    xf = x.reshape(B * CP, H)
    pos_enc, ch_enc = _enc_tables(channels, P, H, C)
    out = _sc_add(xf, pos_enc, ch_enc, P, C)
    return out.reshape(x.shape)


# speedup vs baseline: 3.7992x; 1.0447x over previous
import functools

import jax
import jax.numpy as jnp
from jax.experimental import pallas as pl
from jax.experimental.pallas import tpu as pltpu
from jax.experimental.pallas import tpu_sc as plsc


def _make_enc_body(P, H, C, CHPAD):
    half = H // 2
    quarter = half // 2

    def body(ch_ref, pos_ref, che_ref):
        j = jax.lax.broadcasted_iota(jnp.int32, (1, quarter), 1)
        omega = 1.0 / (10000.0 ** (j.astype(jnp.float32) / float(quarter)))

        sub = 1
        while sub * sub < P:
            sub *= 2
        if sub * sub == P:
            t = jax.lax.broadcasted_iota(jnp.int32, (sub, quarter), 0)
            t = t.astype(jnp.float32)
            ang_a = (t * float(sub)) * omega
            ang_b = t * omega
            sa_all, ca_all = jnp.sin(ang_a), jnp.cos(ang_a)
            sb, cb = jnp.sin(ang_b), jnp.cos(ang_b)
            for a in range(sub):
                sa = sa_all[a:a + 1, :]
                ca = ca_all[a:a + 1, :]
                rows = pl.ds(a * sub, sub)
                pos_ref[rows, :quarter] = sa * cb + ca * sb
                pos_ref[rows, quarter:] = ca * cb - sa * sb
        else:
            p = jax.lax.broadcasted_iota(jnp.int32, (P, quarter), 0)
            ang = p.astype(jnp.float32) * omega
            pos_ref[:, :quarter] = jnp.sin(ang)
            pos_ref[:, quarter:] = jnp.cos(ang)

        for c in range(C):
            ch = ch_ref[c].astype(jnp.float32)
            ang_c = ch * omega
            che_ref[c:c + 1, :quarter] = jnp.sin(ang_c)
            che_ref[c:c + 1, quarter:] = jnp.cos(ang_c)
        if CHPAD > C:
            che_ref[pl.ds(C, CHPAD - C), :] = jnp.zeros(
                (CHPAD - C, half), jnp.float32)

    return body


def _enc_tables(channels, P, H, C):
    CHPAD = 16
    half = H // 2
    return pl.pallas_call(
        _make_enc_body(P, H, C, CHPAD),
        in_specs=[pl.BlockSpec(memory_space=pltpu.SMEM)],
        out_shape=[
            jax.ShapeDtypeStruct((P, half), jnp.float32),
            jax.ShapeDtypeStruct((CHPAD, half), jnp.float32),
        ],
    )(channels)


def _sc_add(xf, pos_enc, ch_enc, P, C):
    R, H = xf.shape
    half = H // 2
    info = plsc.get_sparse_core_info()
    NW = info.num_cores * info.num_subcores
    PW = P // NW
    S = R // P
    mesh = plsc.VectorSubcoreMesh(core_axis_name="c", subcore_axis_name="s")

    NBUF = 4
    assert S % NBUF == 0

    @functools.partial(
        pl.kernel,
        out_type=jax.ShapeDtypeStruct((R, H), jnp.float32),
        mesh=mesh,
        scratch_types=[
            pltpu.VMEM((PW, half), jnp.float32),
            pltpu.VMEM((16, half), jnp.float32),
            pltpu.VMEM((NBUF, PW, H), jnp.float32),
        ] + [pltpu.SemaphoreType.DMA] * (2 * NBUF),
    )
    def run(x_hbm, pos_hbm, ch_hbm, out_hbm, pos_v, ch_v, xt, *sems):
        in_sem, out_sem = sems[:NBUF], sems[NBUF:]
        cid = jax.lax.axis_index("c")
        sid = jax.lax.axis_index("s")
        wid = sid * info.num_cores + cid
        p0 = wid * PW
        pltpu.sync_copy(pos_hbm.at[pl.ds(p0, PW)], pos_v)
        pltpu.sync_copy(ch_hbm, ch_v)

        def in_copy(s, b):
            return pltpu.make_async_copy(
                x_hbm.at[pl.ds(s * P + p0, PW)], xt.at[b], in_sem[b])

        def out_copy(s, b):
            return pltpu.make_async_copy(
                xt.at[b], out_hbm.at[pl.ds(s * P + p0, PW)], out_sem[b])

        for b in range(NBUF - 2):
            in_copy(b, b).start()

        def outer(g, carry):
            for b in range(NBUF):
                s = g + b
                nxt = s + 2
                bn = (b + 2) % NBUF

                @pl.when(s >= 2)
                def _():
                    out_copy(s - 2, bn).wait()

                @pl.when(nxt < S)
                def _():
                    in_copy(nxt, bn).start()

                in_copy(s, b).wait()
                c = jax.lax.rem(s, C)

                @plsc.parallel_loop(0, PW, unroll=4)
                def _row(r):
                    for k in range(half // 16):
                        sl = pl.ds(k * 16, 16)
                        xt[b, r, sl] = xt[b, r, sl] + ch_v[c, sl]
                    for k in range(half // 16):
                        sl_x = pl.ds(half + k * 16, 16)
                        sl_e = pl.ds(k * 16, 16)
                        xt[b, r, sl_x] = xt[b, r, sl_x] + pos_v[r, sl_e]

                out_copy(s, b).start()
            return carry

        jax.lax.fori_loop(0, S // NBUF, lambda i, c: outer(i * NBUF, c), 0)
        for s_tail in (S - 2, S - 1):
            out_copy(s_tail, s_tail % NBUF).wait()

    return run(xf, pos_enc, ch_enc)


def kernel(x, channels):
    B, CP, H = x.shape
    C = channels.shape[0]
    if not C:
        return x
    P = CP // C
    xf = x.reshape(B * CP, H)
    pos_enc, ch_enc = _enc_tables(channels, P, H, C)
    out = _sc_add(xf, pos_enc, ch_enc, P, C)
    return out.reshape(x.shape)


# speedup vs baseline: 3.9610x; 1.0426x over previous
import functools

import jax
import jax.numpy as jnp
from jax.experimental import pallas as pl
from jax.experimental.pallas import tpu as pltpu
from jax.experimental.pallas import tpu_sc as plsc


def _make_enc_body(P, H, C, CHPAD):
    half = H // 2
    quarter = half // 2

    def body(ch_ref, pos_ref, che_ref):
        j = jax.lax.broadcasted_iota(jnp.int32, (1, quarter), 1)
        omega = 1.0 / (10000.0 ** (j.astype(jnp.float32) / float(quarter)))

        sub = 1
        while sub * sub < P:
            sub *= 2
        if sub * sub == P:
            t = jax.lax.broadcasted_iota(jnp.int32, (sub, quarter), 0)
            t = t.astype(jnp.float32)
            ang_a = (t * float(sub)) * omega
            ang_b = t * omega
            sa_all, ca_all = jnp.sin(ang_a), jnp.cos(ang_a)
            sb, cb = jnp.sin(ang_b), jnp.cos(ang_b)
            for a in range(sub):
                sa = sa_all[a:a + 1, :]
                ca = ca_all[a:a + 1, :]
                rows = pl.ds(a * sub, sub)
                pos_ref[rows, :quarter] = sa * cb + ca * sb
                pos_ref[rows, quarter:] = ca * cb - sa * sb
        else:
            p = jax.lax.broadcasted_iota(jnp.int32, (P, quarter), 0)
            ang = p.astype(jnp.float32) * omega
            pos_ref[:, :quarter] = jnp.sin(ang)
            pos_ref[:, quarter:] = jnp.cos(ang)

        for c in range(C):
            ch = ch_ref[c].astype(jnp.float32)
            ang_c = ch * omega
            che_ref[c:c + 1, :quarter] = jnp.sin(ang_c)
            che_ref[c:c + 1, quarter:] = jnp.cos(ang_c)
        if CHPAD > C:
            che_ref[pl.ds(C, CHPAD - C), :] = jnp.zeros(
                (CHPAD - C, half), jnp.float32)

    return body


def _enc_tables(channels, P, H, C):
    CHPAD = 16
    half = H // 2
    return pl.pallas_call(
        _make_enc_body(P, H, C, CHPAD),
        in_specs=[pl.BlockSpec(memory_space=pltpu.SMEM)],
        out_shape=[
            jax.ShapeDtypeStruct((P, half), jnp.float32),
            jax.ShapeDtypeStruct((CHPAD, half), jnp.float32),
        ],
    )(channels)


def _sc_add(xf, pos_enc, ch_enc, P, C):
    R, H = xf.shape
    half = H // 2
    info = plsc.get_sparse_core_info()
    NW = info.num_cores * info.num_subcores
    PW = P // NW
    S = R // P
    mesh = plsc.VectorSubcoreMesh(core_axis_name="c", subcore_axis_name="s")

    NBUF = 4
    assert S % NBUF == 0

    @functools.partial(
        pl.kernel,
        out_type=jax.ShapeDtypeStruct((R, H), jnp.float32),
        mesh=mesh,
        scratch_types=[
            pltpu.VMEM((PW, half), jnp.float32),
            pltpu.VMEM((16, half), jnp.float32),
            pltpu.VMEM((NBUF, PW, H), jnp.float32),
        ] + [pltpu.SemaphoreType.DMA] * (2 * NBUF),
    )
    def run(x_hbm, pos_hbm, ch_hbm, out_hbm, pos_v, ch_v, xt, *sems):
        in_sem, out_sem = sems[:NBUF], sems[NBUF:]
        cid = jax.lax.axis_index("c")
        sid = jax.lax.axis_index("s")
        wid = sid * info.num_cores + cid
        p0 = wid * PW
        pltpu.sync_copy(pos_hbm.at[pl.ds(p0, PW)], pos_v)
        pltpu.sync_copy(ch_hbm, ch_v)

        def in_copy(s, b):
            return pltpu.make_async_copy(
                x_hbm.at[pl.ds(s * P + p0, PW)], xt.at[b], in_sem[b])

        def out_copy(s, b):
            return pltpu.make_async_copy(
                xt.at[b], out_hbm.at[pl.ds(s * P + p0, PW)], out_sem[b])

        for b in range(NBUF - 2):
            in_copy(b, b).start()

        def outer(g, carry):
            for b in range(NBUF):
                s = g + b
                nxt = s + 2
                bn = (b + 2) % NBUF

                @pl.when(s >= 2)
                def _():
                    out_copy(s - 2, bn).wait()

                @pl.when(nxt < S)
                def _():
                    in_copy(nxt, bn).start()

                in_copy(s, b).wait()
                c = jax.lax.rem(s, C)
                chvs = [ch_v[c, pl.ds(k * 16, 16)] for k in range(half // 16)]

                @plsc.parallel_loop(0, PW, unroll=8)
                def _row(r):
                    for k in range(half // 16):
                        sl = pl.ds(k * 16, 16)
                        xt[b, r, sl] = xt[b, r, sl] + chvs[k]
                    for k in range(half // 16):
                        sl_x = pl.ds(half + k * 16, 16)
                        sl_e = pl.ds(k * 16, 16)
                        xt[b, r, sl_x] = xt[b, r, sl_x] + pos_v[r, sl_e]

                out_copy(s, b).start()
            return carry

        jax.lax.fori_loop(0, S // NBUF, lambda i, c: outer(i * NBUF, c), 0)
        for s_tail in (S - 2, S - 1):
            out_copy(s_tail, s_tail % NBUF).wait()

    return run(xf, pos_enc, ch_enc)


def kernel(x, channels):
    B, CP, H = x.shape
    C = channels.shape[0]
    if not C:
        return x
    P = CP // C
    xf = x.reshape(B * CP, H)
    pos_enc, ch_enc = _enc_tables(channels, P, H, C)
    out = _sc_add(xf, pos_enc, ch_enc, P, C)
    return out.reshape(x.shape)


# speedup vs baseline: 5.4893x; 1.3858x over previous
import jax
import jax.numpy as jnp
from jax.experimental import pallas as pl
from jax.experimental.pallas import tpu as pltpu


def _make_body(P, H, C, chunks):
    half = H // 2
    quarter = half // 2

    def body(ch_ref, x_ref, o_ref, pos_ref):
        i = pl.program_id(0)
        j = jax.lax.broadcasted_iota(jnp.int32, (1, quarter), 1)
        omega = 1.0 / (10000.0 ** (j.astype(jnp.float32) / float(quarter)))

        sub = 1
        while sub * sub < P:
            sub *= 2

        @pl.when(i == 0)
        def _():
            if sub * sub == P:
                t = jax.lax.broadcasted_iota(jnp.int32, (sub, quarter), 0)
                t = t.astype(jnp.float32)
                ang_a = (t * float(sub)) * omega
                ang_b = t * omega
                sa_all, ca_all = jnp.sin(ang_a), jnp.cos(ang_a)
                sb, cb = jnp.sin(ang_b), jnp.cos(ang_b)
                for a in range(sub):
                    sa = sa_all[a:a + 1, :]
                    ca = ca_all[a:a + 1, :]
                    rows = pl.ds(a * sub, sub)
                    pos_ref[rows, :quarter] = sa * cb + ca * sb
                    pos_ref[rows, quarter:] = ca * cb - sa * sb
            else:
                p = jax.lax.broadcasted_iota(jnp.int32, (P, quarter), 0)
                ang = p.astype(jnp.float32) * omega
                pos_ref[:, :quarter] = jnp.sin(ang)
                pos_ref[:, quarter:] = jnp.cos(ang)

        for k in range(chunks):
            c = (i * chunks + k) % C
            ch = ch_ref[c].astype(jnp.float32)
            ang_c = ch * omega
            ch_row = jnp.concatenate([jnp.sin(ang_c), jnp.cos(ang_c)], axis=1)
            rows = pl.ds(k * P, P)
            xb = x_ref[0, rows, :]
            o_ref[0, rows, :half] = xb[:, :half] + ch_row
            o_ref[0, rows, half:] = xb[:, half:] + pos_ref[:, :]

    return body


def kernel(x, channels):
    B, CP, H = x.shape
    C = channels.shape[0]
    if not C:
        return x
    P = CP // C
    chunks = 5
    rows = chunks * P
    n = (B * CP) // rows
    xf = x.reshape(n, rows, H)
    out = pl.pallas_call(
        _make_body(P, H, C, chunks),
        grid=(n,),
        in_specs=[
            pl.BlockSpec(memory_space=pltpu.SMEM),
            pl.BlockSpec((1, rows, H), lambda i: (i, 0, 0)),
        ],
        out_specs=pl.BlockSpec((1, rows, H), lambda i: (i, 0, 0)),
        out_shape=jax.ShapeDtypeStruct(xf.shape, x.dtype),
        scratch_shapes=[pltpu.VMEM((P, H // 2), jnp.float32)],
    )(channels, xf)
    return out.reshape(x.shape)
